# baseline (speedup 1.0000x reference)
import jax
import jax.numpy as jnp
from jax.experimental import pallas as pl

NUM_Q = 8
K = 1024
DIM = 64
COMMIT_W = 1.0
BLK = 1024
NSPLIT = 4
SUB = BLK // NSPLIT


def _mm(a, b, dims):
    return jax.lax.dot_general(a, b, (dims, ((), ())),
                               preferred_element_type=jnp.float32)


def _rvq_kernel(y_ref, cb_ref, yhat_ref, idx_ref, ssq_ref):
    i = pl.program_id(0)

    @pl.when(i == 0)
    def _init():
        ssq_ref[...] = jnp.zeros_like(ssq_ref)

    lane_iota = jax.lax.broadcasted_iota(jnp.int32, (SUB, K), 1)
    q_iota = jax.lax.broadcasted_iota(jnp.int32, (1, NUM_Q), 1)
    ys = [y_ref[h * SUB:(h + 1) * SUB, :] for h in range(NSPLIT)]
    res = list(ys)
    idx_cols = [[] for _ in range(NSPLIT)]
    ssq_acc = jnp.zeros((1, NUM_Q), jnp.float32)
    for qi in range(NUM_Q):
        cb = cb_ref[qi]
        s1 = cb.astype(jnp.bfloat16)
        r1 = cb - s1.astype(jnp.float32)
        s2 = r1.astype(jnp.bfloat16)
        s3 = (r1 - s2.astype(jnp.float32)).astype(jnp.bfloat16)
        c2 = jnp.sum(cb * cb, axis=1)[None, :]
        layer_ssq = 0.0
        for h in range(NSPLIT):
            r = res[h]
            x2 = jnp.sum(r * r, axis=1, keepdims=True)
            xc = _mm(r.astype(jnp.bfloat16), s1, ((1,), (1,)))
            d = x2 - 2.0 * xc + c2
            dmin = jnp.min(d, axis=1, keepdims=True)
            idx = jnp.min(jnp.where(d == dmin, lane_iota, K),
                          axis=1, keepdims=True)
            onehot = (lane_iota == idx).astype(jnp.bfloat16)
            q = ((_mm(onehot, s1, ((1,), (0,)))
                  + _mm(onehot, s2, ((1,), (0,))))
                 + _mm(onehot, s3, ((1,), (0,))))
            r = r - q
            res[h] = r
            layer_ssq = layer_ssq + jnp.sum(r * r)
            idx_cols[h].append(idx)
        ssq_acc = ssq_acc + jnp.where(q_iota == qi, layer_ssq, 0.0)
    for h in range(NSPLIT):
        yhat_ref[h * SUB:(h + 1) * SUB, :] = ys[h] - res[h]
        idx_ref[h * SUB:(h + 1) * SUB, :] = jnp.concatenate(idx_cols[h],
                                                            axis=1)
    ssq_ref[...] += ssq_acc


def kernel(y, codebooks):
    b, _ = y.shape
    grid = (b // BLK,)
    cb_spec = pl.BlockSpec((NUM_Q, K, DIM), lambda i: (0, 0, 0))
    yhat, idx, ssq = pl.pallas_call(
        _rvq_kernel,
        grid=grid,
        in_specs=[pl.BlockSpec((BLK, DIM), lambda i: (i, 0)),
                  cb_spec],
        out_specs=[
            pl.BlockSpec((BLK, DIM), lambda i: (i, 0)),
            pl.BlockSpec((BLK, NUM_Q), lambda i: (i, 0)),
            pl.BlockSpec((1, NUM_Q), lambda i: (0, 0)),
        ],
        out_shape=[
            jax.ShapeDtypeStruct((b, DIM), jnp.float32),
            jax.ShapeDtypeStruct((b, NUM_Q), jnp.int32),
            jax.ShapeDtypeStruct((1, NUM_Q), jnp.float32),
        ],
    )(y, codebooks)
    losses_per_layer = COMMIT_W * (ssq[0] / (b * DIM))
    loss_vq = jnp.mean(losses_per_layer)
    return yhat, idx, loss_vq, losses_per_layer


# speedup vs baseline: 1.2450x; 1.2450x over previous
import jax
import jax.numpy as jnp
from jax.experimental import pallas as pl
from jax.experimental.pallas import tpu as pltpu

NUM_Q = 8
K = 1024
DIM = 64
COMMIT_W = 1.0
BLK = 1024
NSPLIT = 2
SUB = BLK // NSPLIT


def _mm(a, b, dims):
    return jax.lax.dot_general(a, b, (dims, ((), ())),
                               preferred_element_type=jnp.float32)


def _rvq_kernel(y_ref, cb_ref, yhat_ref, idx_ref, ssq_ref,
                s1m2_ref, s1_ref, s2_ref, s3_ref, c2_ref):
    i = pl.program_id(0)

    @pl.when(i == 0)
    def _init():
        ssq_ref[...] = jnp.zeros_like(ssq_ref)
        for qi in range(NUM_Q):
            cb = cb_ref[qi]
            s1 = cb.astype(jnp.bfloat16)
            r1 = cb - s1.astype(jnp.float32)
            s2 = r1.astype(jnp.bfloat16)
            s3 = (r1 - s2.astype(jnp.float32)).astype(jnp.bfloat16)
            s1_ref[qi] = s1
            s1m2_ref[qi] = s1 * jnp.bfloat16(-2.0)
            s2_ref[qi] = s2
            s3_ref[qi] = s3
            c2_ref[qi] = jnp.sum(cb * cb, axis=1)[None, :]

    lane_iota = jax.lax.broadcasted_iota(jnp.int32, (SUB, K), 1)
    q_iota = jax.lax.broadcasted_iota(jnp.int32, (1, NUM_Q), 1)
    ys = [y_ref[h * SUB:(h + 1) * SUB, :] for h in range(NSPLIT)]
    res = list(ys)
    idx_cols = [[] for _ in range(NSPLIT)]
    ssq_acc = jnp.zeros((1, NUM_Q), jnp.float32)
    for qi in range(NUM_Q):
        s1m2, s1, s2, s3 = (s1m2_ref[qi], s1_ref[qi], s2_ref[qi],
                            s3_ref[qi])
        c2 = c2_ref[qi]
        layer_ssq = 0.0
        for h in range(NSPLIT):
            r = res[h]
            x2 = jnp.sum(r * r, axis=1, keepdims=True)
            xcm2 = _mm(r.astype(jnp.bfloat16), s1m2, ((1,), (1,)))
            d = x2 + xcm2 + c2
            dmin = jnp.min(d, axis=1, keepdims=True)
            idx = jnp.min(jnp.where(d == dmin, lane_iota, K),
                          axis=1, keepdims=True)
            onehot = (lane_iota == idx).astype(jnp.bfloat16)
            q = ((_mm(onehot, s1, ((1,), (0,)))
                  + _mm(onehot, s2, ((1,), (0,))))
                 + _mm(onehot, s3, ((1,), (0,))))
            r = r - q
            res[h] = r
            layer_ssq = layer_ssq + jnp.sum(r * r)
            idx_cols[h].append(idx)
        ssq_acc = ssq_acc + jnp.where(q_iota == qi, layer_ssq, 0.0)
    for h in range(NSPLIT):
        yhat_ref[h * SUB:(h + 1) * SUB, :] = ys[h] - res[h]
        idx_ref[h * SUB:(h + 1) * SUB, :] = jnp.concatenate(idx_cols[h],
                                                            axis=1)
    ssq_ref[...] += ssq_acc


def kernel(y, codebooks):
    b, _ = y.shape
    grid = (b // BLK,)
    cb_spec = pl.BlockSpec((NUM_Q, K, DIM), lambda i: (0, 0, 0))
    yhat, idx, ssq = pl.pallas_call(
        _rvq_kernel,
        grid=grid,
        in_specs=[pl.BlockSpec((BLK, DIM), lambda i: (i, 0)),
                  cb_spec],
        out_specs=[
            pl.BlockSpec((BLK, DIM), lambda i: (i, 0)),
            pl.BlockSpec((BLK, NUM_Q), lambda i: (i, 0)),
            pl.BlockSpec((1, NUM_Q), lambda i: (0, 0)),
        ],
        out_shape=[
            jax.ShapeDtypeStruct((b, DIM), jnp.float32),
            jax.ShapeDtypeStruct((b, NUM_Q), jnp.int32),
            jax.ShapeDtypeStruct((1, NUM_Q), jnp.float32),
        ],
        scratch_shapes=[
            pltpu.VMEM((NUM_Q, K, DIM), jnp.bfloat16),
            pltpu.VMEM((NUM_Q, K, DIM), jnp.bfloat16),
            pltpu.VMEM((NUM_Q, K, DIM), jnp.bfloat16),
            pltpu.VMEM((NUM_Q, K, DIM), jnp.bfloat16),
            pltpu.VMEM((NUM_Q, 1, K), jnp.float32),
        ],
    )(y, codebooks)
    losses_per_layer = COMMIT_W * (ssq[0] / (b * DIM))
    loss_vq = jnp.mean(losses_per_layer)
    return yhat, idx, loss_vq, losses_per_layer


# speedup vs baseline: 1.3125x; 1.0542x over previous
import jax
import jax.numpy as jnp
from jax.experimental import pallas as pl
from jax.experimental.pallas import tpu as pltpu

NUM_Q = 8
K = 1024
DIM = 64
COMMIT_W = 1.0
BLK = 1024
NSPLIT = 2
SUB = BLK // NSPLIT


def _mm(a, b, dims):
    return jax.lax.dot_general(a, b, (dims, ((), ())),
                               preferred_element_type=jnp.float32)


def _rvq_kernel(y_ref, cb_ref, yhat_ref, idx_ref, ssq_ref,
                s1m2_ref, s123_ref, c2_ref):
    i = pl.program_id(0)

    @pl.when(i == 0)
    def _init():
        ssq_ref[...] = jnp.zeros_like(ssq_ref)
        for qi in range(NUM_Q):
            cb = cb_ref[qi]
            s1 = cb.astype(jnp.bfloat16)
            r1 = cb - s1.astype(jnp.float32)
            s2 = r1.astype(jnp.bfloat16)
            s3 = (r1 - s2.astype(jnp.float32)).astype(jnp.bfloat16)
            s1m2_ref[qi] = s1 * jnp.bfloat16(-2.0)
            s123_ref[qi] = jnp.concatenate([s1, s2, s3], axis=1)
            c2_ref[qi] = jnp.sum(cb * cb, axis=1)[None, :]

    lane_iota = jax.lax.broadcasted_iota(jnp.int32, (SUB, K), 1)
    q_iota = jax.lax.broadcasted_iota(jnp.int32, (1, NUM_Q), 1)
    ys = [y_ref[h * SUB:(h + 1) * SUB, :] for h in range(NSPLIT)]
    res = list(ys)
    idx_cols = [[] for _ in range(NSPLIT)]
    ssq_acc = jnp.zeros((1, NUM_Q), jnp.float32)
    for qi in range(NUM_Q):
        s1m2, s123 = s1m2_ref[qi], s123_ref[qi]
        c2 = c2_ref[qi]
        layer_ssq = 0.0
        for h in range(NSPLIT):
            r = res[h]
            x2 = jnp.sum(r * r, axis=1, keepdims=True)
            xcm2 = _mm(r.astype(jnp.bfloat16), s1m2, ((1,), (1,)))
            d = x2 + xcm2 + c2
            dmin = jnp.min(d, axis=1, keepdims=True)
            idx = jnp.min(jnp.where(d == dmin, lane_iota, K),
                          axis=1, keepdims=True)
            onehot = (lane_iota == idx).astype(jnp.bfloat16)
            q3 = _mm(onehot, s123, ((1,), (0,)))
            q = ((q3[:, 0:DIM] + q3[:, DIM:2 * DIM])
                 + q3[:, 2 * DIM:3 * DIM])
            r = r - q
            res[h] = r
            layer_ssq = layer_ssq + jnp.sum(r * r)
            idx_cols[h].append(idx)
        ssq_acc = ssq_acc + jnp.where(q_iota == qi, layer_ssq, 0.0)
    for h in range(NSPLIT):
        yhat_ref[h * SUB:(h + 1) * SUB, :] = ys[h] - res[h]
        idx_ref[h * SUB:(h + 1) * SUB, :] = jnp.concatenate(idx_cols[h],
                                                            axis=1)
    ssq_ref[...] += ssq_acc


def kernel(y, codebooks):
    b, _ = y.shape
    grid = (b // BLK,)
    cb_spec = pl.BlockSpec((NUM_Q, K, DIM), lambda i: (0, 0, 0))
    yhat, idx, ssq = pl.pallas_call(
        _rvq_kernel,
        grid=grid,
        in_specs=[pl.BlockSpec((BLK, DIM), lambda i: (i, 0)),
                  cb_spec],
        out_specs=[
            pl.BlockSpec((BLK, DIM), lambda i: (i, 0)),
            pl.BlockSpec((BLK, NUM_Q), lambda i: (i, 0)),
            pl.BlockSpec((1, NUM_Q), lambda i: (0, 0)),
        ],
        out_shape=[
            jax.ShapeDtypeStruct((b, DIM), jnp.float32),
            jax.ShapeDtypeStruct((b, NUM_Q), jnp.int32),
            jax.ShapeDtypeStruct((1, NUM_Q), jnp.float32),
        ],
        scratch_shapes=[
            pltpu.VMEM((NUM_Q, K, DIM), jnp.bfloat16),
            pltpu.VMEM((NUM_Q, K, 3 * DIM), jnp.bfloat16),
            pltpu.VMEM((NUM_Q, 1, K), jnp.float32),
        ],
    )(y, codebooks)
    losses_per_layer = COMMIT_W * (ssq[0] / (b * DIM))
    loss_vq = jnp.mean(losses_per_layer)
    return yhat, idx, loss_vq, losses_per_layer


# speedup vs baseline: 1.4056x; 1.0709x over previous
import jax
import jax.numpy as jnp
from jax.experimental import pallas as pl
from jax.experimental.pallas import tpu as pltpu

NUM_Q = 8
K = 1024
DIM = 64
COMMIT_W = 1.0
BLK = 1024
NSPLIT = 2
SUB = BLK // NSPLIT


def _mm(a, b, dims):
    return jax.lax.dot_general(a, b, (dims, ((), ())),
                               preferred_element_type=jnp.float32)


def _rvq_kernel(y_ref, cb_ref, yhat_ref, idx_ref, ssq_ref,
                s1m2_ref, s123_ref, c2_ref):
    i = pl.program_id(0)

    @pl.when(i == 0)
    def _init():
        ssq_ref[...] = jnp.zeros_like(ssq_ref)
        for qi in range(NUM_Q):
            cb = cb_ref[qi]
            s1 = cb.astype(jnp.bfloat16)
            r1 = cb - s1.astype(jnp.float32)
            s2 = r1.astype(jnp.bfloat16)
            s3 = (r1 - s2.astype(jnp.float32)).astype(jnp.bfloat16)
            s1m2_ref[qi] = s1 * jnp.bfloat16(-2.0)
            s123_ref[qi] = jnp.concatenate([s1, s2, s3], axis=1)
            c2_ref[qi] = jnp.sum(cb * cb, axis=1)[None, :]

    lane_iota = jax.lax.broadcasted_iota(
        jnp.int32, (SUB, K), 1).astype(jnp.float32)
    q_iota = jax.lax.broadcasted_iota(jnp.int32, (1, NUM_Q), 1)
    ys = [y_ref[h * SUB:(h + 1) * SUB, :] for h in range(NSPLIT)]
    res = list(ys)
    idx_cols = [[] for _ in range(NSPLIT)]
    ssq_acc = jnp.zeros((1, NUM_Q), jnp.float32)
    for qi in range(NUM_Q):
        s1m2, s123 = s1m2_ref[qi], s123_ref[qi]
        c2 = c2_ref[qi]
        layer_ssq = 0.0
        for h in range(NSPLIT):
            r = res[h]
            x2 = jnp.sum(r * r, axis=1, keepdims=True)
            if qi > 0:
                ssq_acc = ssq_acc + jnp.where(q_iota == qi - 1,
                                              jnp.sum(x2), 0.0)
            xcm2 = _mm(r.astype(jnp.bfloat16), s1m2, ((1,), (1,)))
            d = x2 + xcm2 + c2
            dmin = jnp.min(d, axis=1, keepdims=True)
            idxf = jnp.min(jnp.where(d == dmin, lane_iota, float(K)),
                           axis=1, keepdims=True)
            onehot = (lane_iota == idxf).astype(jnp.bfloat16)
            q3 = _mm(onehot, s123, ((1,), (0,)))
            q = ((q3[:, 0:DIM] + q3[:, DIM:2 * DIM])
                 + q3[:, 2 * DIM:3 * DIM])
            r = r - q
            res[h] = r
            if qi == NUM_Q - 1:
                layer_ssq = layer_ssq + jnp.sum(r * r)
            idx_cols[h].append(idxf.astype(jnp.int32))
        if qi == NUM_Q - 1:
            ssq_acc = ssq_acc + jnp.where(q_iota == qi, layer_ssq, 0.0)
    for h in range(NSPLIT):
        yhat_ref[h * SUB:(h + 1) * SUB, :] = ys[h] - res[h]
        idx_ref[h * SUB:(h + 1) * SUB, :] = jnp.concatenate(idx_cols[h],
                                                            axis=1)
    ssq_ref[...] += ssq_acc


def kernel(y, codebooks):
    b, _ = y.shape
    grid = (b // BLK,)
    cb_spec = pl.BlockSpec((NUM_Q, K, DIM), lambda i: (0, 0, 0))
    yhat, idx, ssq = pl.pallas_call(
        _rvq_kernel,
        grid=grid,
        in_specs=[pl.BlockSpec((BLK, DIM), lambda i: (i, 0)),
                  cb_spec],
        out_specs=[
            pl.BlockSpec((BLK, DIM), lambda i: (i, 0)),
            pl.BlockSpec((BLK, NUM_Q), lambda i: (i, 0)),
            pl.BlockSpec((1, NUM_Q), lambda i: (0, 0)),
        ],
        out_shape=[
            jax.ShapeDtypeStruct((b, DIM), jnp.float32),
            jax.ShapeDtypeStruct((b, NUM_Q), jnp.int32),
            jax.ShapeDtypeStruct((1, NUM_Q), jnp.float32),
        ],
        scratch_shapes=[
            pltpu.VMEM((NUM_Q, K, DIM), jnp.bfloat16),
            pltpu.VMEM((NUM_Q, K, 3 * DIM), jnp.bfloat16),
            pltpu.VMEM((NUM_Q, 1, K), jnp.float32),
        ],
    )(y, codebooks)
    losses_per_layer = COMMIT_W * (ssq[0] / (b * DIM))
    loss_vq = jnp.mean(losses_per_layer)
    return yhat, idx, loss_vq, losses_per_layer
